# baseline (speedup 1.0000x reference)
import functools

import jax
import jax.numpy as jnp
from jax import lax
from jax.experimental import pallas as pl
from jax.experimental.pallas import tpu as pltpu
from jax.experimental.pallas import tpu_sc as plsc

N_NODES = 10000
N_EDGES = 320000
D_IN = 128
D_OUT = 128
DP = 160

NC = 2
NS = 16
NW = NC * NS
EDGES_PER_W = N_EDGES // NW
CHUNK = 128
NSTREAM = 3
CPS = 26
MAIN_PER_W = NSTREAM * CPS * CHUNK
TAIL = EDGES_PER_W - MAIN_PER_W
ROWS_PER_S = N_NODES // NS
STAGE = 125


_mesh = plsc.VectorSubcoreMesh(
    core_axis_name="c", subcore_axis_name="s", num_cores=NC, num_subcores=NS
)


@functools.partial(
    pl.kernel,
    out_type=jax.ShapeDtypeStruct((NC, N_NODES, DP), jnp.bfloat16),
    mesh=_mesh,
    scratch_types=[
        pltpu.VMEM_SHARED((N_NODES, DP), jnp.bfloat16),
        pltpu.VMEM((CPS, CHUNK), jnp.int32),
        pltpu.VMEM((CPS, CHUNK), jnp.int32),
        pltpu.VMEM((CPS, CHUNK), jnp.int32),
        pltpu.VMEM((CPS, CHUNK), jnp.int32),
        pltpu.VMEM((CPS, CHUNK), jnp.int32),
        pltpu.VMEM((CPS, CHUNK), jnp.int32),
        pltpu.VMEM((TAIL,), jnp.int32),
        pltpu.VMEM((TAIL,), jnp.int32),
        pltpu.VMEM((CHUNK, DP), jnp.bfloat16),
        pltpu.VMEM((CHUNK, DP), jnp.bfloat16),
        pltpu.VMEM((CHUNK, DP), jnp.bfloat16),
        pltpu.VMEM((STAGE, DP), jnp.bfloat16),
        pltpu.SemaphoreType.DMA,
        pltpu.SemaphoreType.DMA,
        pltpu.SemaphoreType.DMA,
        pltpu.SemaphoreType.DMA,
        pltpu.SemaphoreType.DMA,
        pltpu.SemaphoreType.DMA,
    ],
    compiler_params=pltpu.CompilerParams(use_tc_tiling_on_sc=False),
)
def _sc_scatter(x_hbm, col_hbm, row_hbm, colt_hbm, rowt_hbm, zacc_hbm,
                acc_hbm,
                acc_s, col_a, row_a, col_b, row_b, col_c, row_c,
                colt_v, rowt_v, buf_a, buf_b, buf_c, wstage_v,
                sem_ga, sem_gb, sem_gc, sem_sa, sem_sb, sem_sc):
    c = lax.axis_index("c")
    s = lax.axis_index("s")
    g = c * NS + s

    base_rows = s * ROWS_PER_S
    pltpu.sync_copy(zacc_hbm, acc_s.at[pl.ds(base_rows, ROWS_PER_S)])

    plsc.subcore_barrier()

    base_c = g * NSTREAM * CPS
    pltpu.sync_copy(col_hbm.at[pl.ds(base_c, CPS)], col_a)
    pltpu.sync_copy(row_hbm.at[pl.ds(base_c, CPS)], row_a)
    pltpu.sync_copy(col_hbm.at[pl.ds(base_c + CPS, CPS)], col_b)
    pltpu.sync_copy(row_hbm.at[pl.ds(base_c + CPS, CPS)], row_b)
    pltpu.sync_copy(col_hbm.at[pl.ds(base_c + 2 * CPS, CPS)], col_c)
    pltpu.sync_copy(row_hbm.at[pl.ds(base_c + 2 * CPS, CPS)], row_c)
    pltpu.sync_copy(colt_hbm.at[g], colt_v)
    pltpu.sync_copy(rowt_hbm.at[g], rowt_v)

    streams = ((col_a, row_a, buf_a, sem_ga, sem_sa),
               (col_b, row_b, buf_b, sem_gb, sem_sb),
               (col_c, row_c, buf_c, sem_gc, sem_sc))

    for (colv, rowv, buf, sg, ss) in streams:
        pltpu.async_copy(x_hbm.at[colv.at[0]], buf, sg)

    def _pipe(j, carry):
        for (colv, rowv, buf, sg, ss) in streams:
            pltpu.make_async_copy(x_hbm.at[colv.at[j]], buf, sg).wait()
            pltpu.async_copy(buf, acc_s.at[rowv.at[j]], ss, add=True)

        for (colv, rowv, buf, sg, ss) in streams:
            pltpu.make_async_copy(buf, acc_s.at[rowv.at[j]], ss).wait()

            @pl.when(j < CPS - 1)
            def _():
                pltpu.async_copy(x_hbm.at[colv.at[j + 1]], buf, sg)

        return carry

    lax.fori_loop(0, CPS, _pipe, 0)

    buft = buf_a.at[pl.ds(0, TAIL)]
    pltpu.async_copy(x_hbm.at[colt_v], buft, sem_ga).wait()
    pltpu.sync_copy(buft, acc_s.at[rowt_v], add=True)

    plsc.subcore_barrier()

    def _wb(t, carry):
        r0 = base_rows + t * STAGE
        pltpu.sync_copy(acc_s.at[pl.ds(r0, STAGE)], wstage_v)
        pltpu.sync_copy(wstage_v, acc_hbm.at[c, pl.ds(r0, STAGE)])
        return carry

    lax.fori_loop(0, ROWS_PER_S // STAGE, _wb, 0)


_TC_BLK = 2000


def _tc_body(acc_ref, x_ref, w_ref, b_ref, out_ref):
    feats = (acc_ref[0, :, :D_IN].astype(jnp.float32)
             + acc_ref[1, :, :D_IN].astype(jnp.float32) + x_ref[...])
    dot = lax.dot_general(feats, w_ref[...], (((1,), (1,)), ((), ())),
                          preferred_element_type=jnp.float32)
    degcol = (acc_ref[0, :, D_IN:D_IN + 1].astype(jnp.float32)
              + acc_ref[1, :, D_IN:D_IN + 1].astype(jnp.float32)) + 1.0
    out_ref[...] = dot + degcol * b_ref[...]


def _tc_matmul(acc, x, W, b2d):
    return pl.pallas_call(
        _tc_body,
        out_shape=jax.ShapeDtypeStruct((N_NODES, D_OUT), jnp.float32),
        grid=(N_NODES // _TC_BLK,),
        in_specs=[
            pl.BlockSpec((NC, _TC_BLK, DP), lambda i: (0, i, 0)),
            pl.BlockSpec((_TC_BLK, D_IN), lambda i: (i, 0)),
            pl.BlockSpec((D_OUT, D_IN), lambda i: (0, 0)),
            pl.BlockSpec((1, D_OUT), lambda i: (0, 0)),
        ],
        out_specs=pl.BlockSpec((_TC_BLK, D_OUT), lambda i: (i, 0)),
    )(acc, x, W, b2d)


def kernel(x, edge_index, W, b):
    ei = edge_index.astype(jnp.int32)
    row_w = ei[0].reshape(NW, EDGES_PER_W)
    col_w = ei[1].reshape(NW, EDGES_PER_W)
    row2d = row_w[:, :MAIN_PER_W].reshape(NW * NSTREAM * CPS, CHUNK)
    col2d = col_w[:, :MAIN_PER_W].reshape(NW * NSTREAM * CPS, CHUNK)
    rowt = row_w[:, MAIN_PER_W:]
    colt = col_w[:, MAIN_PER_W:]
    xaug = jnp.concatenate(
        [x.astype(jnp.bfloat16),
         jnp.ones((N_NODES, 1), jnp.bfloat16),
         jnp.zeros((N_NODES, DP - D_IN - 1), jnp.bfloat16)], axis=1)
    zacc = jnp.zeros((ROWS_PER_S, DP), jnp.bfloat16)
    acc = _sc_scatter(xaug, col2d, row2d, colt, rowt, zacc)
    return _tc_matmul(acc, x, W, b.reshape(1, D_OUT))


# speedup vs baseline: 1.2214x; 1.2214x over previous
import functools

import jax
import jax.numpy as jnp
from jax import lax
from jax.experimental import pallas as pl
from jax.experimental.pallas import tpu as pltpu
from jax.experimental.pallas import tpu_sc as plsc

N_NODES = 10000
N_EDGES = 320000
D_IN = 128
D_OUT = 128
DDEG = 8

NC = 2
NS = 16
NW = NC * NS
EDGES_PER_W = N_EDGES // NW
CHUNK = 128
NSTREAM = 3
CPS = 26
MAIN_PER_W = NSTREAM * CPS * CHUNK
TAIL = EDGES_PER_W - MAIN_PER_W
ROWS_PER_S = N_NODES // NS
STAGE = 125


_mesh = plsc.VectorSubcoreMesh(
    core_axis_name="c", subcore_axis_name="s", num_cores=NC, num_subcores=NS
)


@functools.partial(
    pl.kernel,
    out_type=(
        jax.ShapeDtypeStruct((NC, N_NODES, D_IN), jnp.bfloat16),
        jax.ShapeDtypeStruct((NC, N_NODES, DDEG), jnp.float32),
    ),
    mesh=_mesh,
    scratch_types=[
        pltpu.VMEM_SHARED((N_NODES, D_IN), jnp.bfloat16),
        pltpu.VMEM_SHARED((N_NODES, DDEG), jnp.float32),
        pltpu.VMEM((CPS, CHUNK), jnp.int32),
        pltpu.VMEM((CPS, CHUNK), jnp.int32),
        pltpu.VMEM((CPS, CHUNK), jnp.int32),
        pltpu.VMEM((CPS, CHUNK), jnp.int32),
        pltpu.VMEM((CPS, CHUNK), jnp.int32),
        pltpu.VMEM((CPS, CHUNK), jnp.int32),
        pltpu.VMEM((TAIL,), jnp.int32),
        pltpu.VMEM((TAIL,), jnp.int32),
        pltpu.VMEM((CHUNK, D_IN), jnp.bfloat16),
        pltpu.VMEM((CHUNK, D_IN), jnp.bfloat16),
        pltpu.VMEM((CHUNK, D_IN), jnp.bfloat16),
        pltpu.VMEM((CHUNK, DDEG), jnp.float32),
        pltpu.VMEM((STAGE, D_IN), jnp.bfloat16),
        pltpu.VMEM((ROWS_PER_S, DDEG), jnp.float32),
        pltpu.SemaphoreType.DMA,
        pltpu.SemaphoreType.DMA,
        pltpu.SemaphoreType.DMA,
        pltpu.SemaphoreType.DMA,
        pltpu.SemaphoreType.DMA,
        pltpu.SemaphoreType.DMA,
        pltpu.SemaphoreType.DMA,
    ],
    compiler_params=pltpu.CompilerParams(use_tc_tiling_on_sc=False),
)
def _sc_scatter(x_hbm, col_hbm, row_hbm, colt_hbm, rowt_hbm,
                ones_hbm, zacc_hbm, zdeg_hbm, acc_hbm, deg_hbm,
                acc_s, deg_s, col_a, row_a, col_b, row_b, col_c, row_c,
                colt_v, rowt_v, buf_a, buf_b, buf_c, ones_v,
                wstage_v, dstage_v,
                sem_ga, sem_gb, sem_gc, sem_sa, sem_sb, sem_sc, sem_d):
    c = lax.axis_index("c")
    s = lax.axis_index("s")
    g = c * NS + s

    base_rows = s * ROWS_PER_S
    base_c = g * NSTREAM * CPS
    zacc_sl = acc_s.at[pl.ds(base_rows, ROWS_PER_S)]
    zdeg_sl = deg_s.at[pl.ds(base_rows, ROWS_PER_S)]
    pltpu.async_copy(zacc_hbm, zacc_sl, sem_sa)
    pltpu.async_copy(zdeg_hbm, zdeg_sl, sem_sb)
    pltpu.async_copy(col_hbm.at[pl.ds(base_c, CPS)], col_a, sem_ga)
    pltpu.async_copy(row_hbm.at[pl.ds(base_c, CPS)], row_a, sem_ga)
    pltpu.async_copy(col_hbm.at[pl.ds(base_c + CPS, CPS)], col_b, sem_gb)
    pltpu.async_copy(row_hbm.at[pl.ds(base_c + CPS, CPS)], row_b, sem_gb)
    pltpu.async_copy(col_hbm.at[pl.ds(base_c + 2 * CPS, CPS)], col_c, sem_gc)
    pltpu.async_copy(row_hbm.at[pl.ds(base_c + 2 * CPS, CPS)], row_c, sem_gc)
    pltpu.async_copy(colt_hbm.at[g], colt_v, sem_sc)
    pltpu.async_copy(rowt_hbm.at[g], rowt_v, sem_sc)
    pltpu.sync_copy(ones_hbm, ones_v)
    pltpu.make_async_copy(zacc_hbm, zacc_sl, sem_sa).wait()
    pltpu.make_async_copy(zdeg_hbm, zdeg_sl, sem_sb).wait()
    pltpu.make_async_copy(col_hbm.at[pl.ds(base_c, CPS)], col_a, sem_ga).wait()
    pltpu.make_async_copy(row_hbm.at[pl.ds(base_c, CPS)], row_a, sem_ga).wait()
    pltpu.make_async_copy(col_hbm.at[pl.ds(base_c, CPS)], col_b, sem_gb).wait()
    pltpu.make_async_copy(row_hbm.at[pl.ds(base_c, CPS)], row_b, sem_gb).wait()
    pltpu.make_async_copy(col_hbm.at[pl.ds(base_c, CPS)], col_c, sem_gc).wait()
    pltpu.make_async_copy(row_hbm.at[pl.ds(base_c, CPS)], row_c, sem_gc).wait()
    pltpu.make_async_copy(colt_hbm.at[g], colt_v, sem_sc).wait()
    pltpu.make_async_copy(rowt_hbm.at[g], rowt_v, sem_sc).wait()

    plsc.subcore_barrier()

    streams = ((col_a, row_a, buf_a, sem_ga, sem_sa),
               (col_b, row_b, buf_b, sem_gb, sem_sb),
               (col_c, row_c, buf_c, sem_gc, sem_sc))

    for (colv, rowv, buf, sg, ss) in streams:
        pltpu.async_copy(x_hbm.at[colv.at[0]], buf, sg)

    def _pipe(j, carry):
        for (colv, rowv, buf, sg, ss) in streams:
            pltpu.async_copy(ones_v, deg_s.at[rowv.at[j]], sem_d, add=True)
            pltpu.make_async_copy(x_hbm.at[colv.at[j]], buf, sg).wait()
            pltpu.async_copy(buf, acc_s.at[rowv.at[j]], ss, add=True)

        for (colv, rowv, buf, sg, ss) in streams:
            pltpu.make_async_copy(buf, acc_s.at[rowv.at[j]], ss).wait()

            @pl.when(j < CPS - 1)
            def _():
                pltpu.async_copy(x_hbm.at[colv.at[j + 1]], buf, sg)

        for (colv, rowv, buf, sg, ss) in streams:
            pltpu.make_async_copy(ones_v, deg_s.at[rowv.at[j]], sem_d).wait()

        return carry

    lax.fori_loop(0, CPS, _pipe, 0)

    onest = ones_v.at[pl.ds(0, TAIL)]
    buft = buf_a.at[pl.ds(0, TAIL)]
    pltpu.async_copy(onest, deg_s.at[rowt_v], sem_d, add=True)
    pltpu.async_copy(x_hbm.at[colt_v], buft, sem_ga).wait()
    pltpu.sync_copy(buft, acc_s.at[rowt_v], add=True)
    pltpu.make_async_copy(onest, deg_s.at[rowt_v], sem_d).wait()

    plsc.subcore_barrier()

    wb_acc = acc_hbm.at[c, pl.ds(base_rows, ROWS_PER_S)]
    wb_deg = deg_hbm.at[c, pl.ds(base_rows, ROWS_PER_S)]
    pltpu.async_copy(acc_s.at[pl.ds(base_rows, ROWS_PER_S)], wb_acc, sem_sa)
    pltpu.async_copy(deg_s.at[pl.ds(base_rows, ROWS_PER_S)], wb_deg, sem_sb)
    pltpu.make_async_copy(acc_s.at[pl.ds(base_rows, ROWS_PER_S)], wb_acc,
                          sem_sa).wait()
    pltpu.make_async_copy(deg_s.at[pl.ds(base_rows, ROWS_PER_S)], wb_deg,
                          sem_sb).wait()


_TC_BLK = 2000


def _tc_body(acc_ref, deg_ref, x_ref, w_ref, b_ref, out_ref):
    srows = (acc_ref[0].astype(jnp.float32) + acc_ref[1].astype(jnp.float32)
             + x_ref[...])
    dot = lax.dot_general(srows, w_ref[...], (((1,), (1,)), ((), ())),
                          preferred_element_type=jnp.float32)
    degcol = (deg_ref[0, :, 0:1] + deg_ref[1, :, 0:1]) + 1.0
    out_ref[...] = dot + degcol * b_ref[...]


def _tc_matmul(acc, deg, x, W, b2d):
    return pl.pallas_call(
        _tc_body,
        out_shape=jax.ShapeDtypeStruct((N_NODES, D_OUT), jnp.float32),
        grid=(N_NODES // _TC_BLK,),
        in_specs=[
            pl.BlockSpec((NC, _TC_BLK, D_IN), lambda i: (0, i, 0)),
            pl.BlockSpec((NC, _TC_BLK, DDEG), lambda i: (0, i, 0)),
            pl.BlockSpec((_TC_BLK, D_IN), lambda i: (i, 0)),
            pl.BlockSpec((D_OUT, D_IN), lambda i: (0, 0)),
            pl.BlockSpec((1, D_OUT), lambda i: (0, 0)),
        ],
        out_specs=pl.BlockSpec((_TC_BLK, D_OUT), lambda i: (i, 0)),
    )(acc, deg, x, W, b2d)


def kernel(x, edge_index, W, b):
    ei = edge_index.astype(jnp.int32)
    row_w = ei[0].reshape(NW, EDGES_PER_W)
    col_w = ei[1].reshape(NW, EDGES_PER_W)
    row2d = row_w[:, :MAIN_PER_W].reshape(NW * NSTREAM * CPS, CHUNK)
    col2d = col_w[:, :MAIN_PER_W].reshape(NW * NSTREAM * CPS, CHUNK)
    rowt = row_w[:, MAIN_PER_W:]
    colt = col_w[:, MAIN_PER_W:]
    xbf = x.astype(jnp.bfloat16)
    ones8 = jnp.ones((CHUNK, DDEG), jnp.float32)
    zacc = jnp.zeros((ROWS_PER_S, D_IN), jnp.bfloat16)
    zdeg = jnp.zeros((ROWS_PER_S, DDEG), jnp.float32)
    acc, deg = _sc_scatter(xbf, col2d, row2d, colt, rowt, ones8, zacc, zdeg)
    return _tc_matmul(acc, deg, x, W, b.reshape(1, D_OUT))
